# baseline (speedup 1.0000x reference)
import functools

import jax
import jax.numpy as jnp
from jax import lax
from jax.experimental import pallas as pl
from jax.experimental.pallas import tpu as pltpu
from jax.experimental.pallas import tpu_sc as plsc

B, T, D = 4, 8192, 1024
NC, NS, L = 2, 16, 16
NW = NC * NS
ROWS_W = T // NW
R = 32

_mesh = plsc.VectorSubcoreMesh(core_axis_name="c", subcore_axis_name="s")


@functools.partial(
    pl.kernel,
    mesh=_mesh,
    out_type=jax.ShapeDtypeStruct((B, T, D), jnp.float32),
    scratch_types=[
        pltpu.VMEM((R, D), jnp.float32),
        pltpu.VMEM((R, D), jnp.float32),
    ],
)
def _sc_add(x_hbm, pos_hbm, out_hbm, pos_v, buf_v):
    wid = lax.axis_index("s") * NC + lax.axis_index("c")
    base = wid * ROWS_W

    def block(i, carry):
        r0 = base + i * R
        pltpu.sync_copy(pos_hbm.at[pl.ds(r0, R)], pos_v)
        for b in range(B):
            pltpu.sync_copy(x_hbm.at[b, pl.ds(r0, R)], buf_v)

            def row(r, c2):
                for c in range(D // L):
                    sl = pl.ds(c * L, L)
                    buf_v[r, sl] = buf_v[r, sl] + pos_v[r, sl]
                return c2

            lax.fori_loop(0, R, row, 0)
            pltpu.sync_copy(buf_v, out_hbm.at[b, pl.ds(r0, R)])
        return carry

    lax.fori_loop(0, ROWS_W // R, block, 0)


def kernel(x, pos_table):
    return _sc_add(x, pos_table)


# speedup vs baseline: 2.0735x; 2.0735x over previous
import functools

import jax
import jax.numpy as jnp
from jax import lax
from jax.experimental import pallas as pl
from jax.experimental.pallas import tpu as pltpu
from jax.experimental.pallas import tpu_sc as plsc

B, T, D = 4, 8192, 1024
NC, NS, L = 2, 16, 16
NW = NC * NS
ROWS_W = T // NW
RB = 4
NB = ROWS_W // RB
RING = 4

_mesh = plsc.VectorSubcoreMesh(core_axis_name="c", subcore_axis_name="s")

_scratch = (
    [pltpu.VMEM((B, RB, D), jnp.float32) for _ in range(RING)]
    + [pltpu.VMEM((RB, D), jnp.float32) for _ in range(RING)]
    + [pltpu.SemaphoreType.DMA] * (3 * RING)
)


@functools.partial(
    pl.kernel,
    mesh=_mesh,
    out_type=jax.ShapeDtypeStruct((B, T, D), jnp.float32),
    scratch_types=_scratch,
)
def _sc_add(x_hbm, pos_hbm, out_hbm, *refs):
    xb = refs[0:RING]
    pb = refs[RING : 2 * RING]
    in_sem = refs[2 * RING : 3 * RING]
    pos_sem = refs[3 * RING : 4 * RING]
    out_sem = refs[4 * RING : 5 * RING]

    wid = lax.axis_index("s") * NC + lax.axis_index("c")
    base = wid * ROWS_W

    def start_in(i, j):
        r0 = base + i * RB
        pltpu.async_copy(x_hbm.at[:, pl.ds(r0, RB)], xb[j], in_sem[j])
        pltpu.async_copy(pos_hbm.at[pl.ds(r0, RB)], pb[j], pos_sem[j])

    def wait_in(j):
        pltpu.make_async_copy(x_hbm.at[:, pl.ds(base, RB)], xb[j], in_sem[j]).wait()
        pltpu.make_async_copy(pos_hbm.at[pl.ds(base, RB)], pb[j], pos_sem[j]).wait()

    def start_out(i, j):
        r0 = base + i * RB
        pltpu.async_copy(xb[j], out_hbm.at[:, pl.ds(r0, RB)], out_sem[j])

    def wait_out(j):
        pltpu.make_async_copy(xb[j], out_hbm.at[:, pl.ds(base, RB)], out_sem[j]).wait()

    start_in(0, 0)
    start_in(1, 1)

    def outer(ii, carry):
        i0 = ii * RING
        for j in range(RING):
            i = i0 + j
            jp = (j + 2) % RING

            @pl.when(i >= 2)
            def _():
                wait_out(jp)

            @pl.when(i + 2 < NB)
            def _():
                start_in(i + 2, jp)

            wait_in(j)

            def col(c, cc):
                sl = pl.ds(c * L, L)
                for r in range(RB):
                    v = pb[j][r, sl]
                    for b in range(B):
                        plsc.addupdate(xb[j].at[b, r, sl], v)
                return cc

            lax.fori_loop(0, D // L, col, 0)
            start_out(i, j)
        return carry

    lax.fori_loop(0, NB // RING, outer, 0)

    wait_out((NB - 2) % RING)
    wait_out((NB - 1) % RING)


def kernel(x, pos_table):
    return _sc_add(x, pos_table)


# speedup vs baseline: 2.0765x; 1.0014x over previous
import functools

import jax
import jax.numpy as jnp
from jax import lax
from jax.experimental import pallas as pl
from jax.experimental.pallas import tpu as pltpu
from jax.experimental.pallas import tpu_sc as plsc

B, T, D = 4, 8192, 1024
NC, NS, L = 2, 16, 16
NW = NC * NS
ROWS_W = T // NW
RB = 4
NB = ROWS_W // RB
RING = 4

_mesh = plsc.VectorSubcoreMesh(core_axis_name="c", subcore_axis_name="s")

_scratch = (
    [pltpu.VMEM((B, RB, D), jnp.float32) for _ in range(RING)]
    + [pltpu.VMEM((RB, D), jnp.float32) for _ in range(RING)]
    + [pltpu.SemaphoreType.DMA] * (3 * RING)
)


@functools.partial(
    pl.kernel,
    mesh=_mesh,
    out_type=jax.ShapeDtypeStruct((B, T, D), jnp.float32),
    scratch_types=_scratch,
)
def _sc_add(x_hbm, pos_hbm, out_hbm, *refs):
    xb = refs[0:RING]
    pb = refs[RING : 2 * RING]
    in_sem = refs[2 * RING : 3 * RING]
    pos_sem = refs[3 * RING : 4 * RING]
    out_sem = refs[4 * RING : 5 * RING]

    wid = lax.axis_index("s") * NC + lax.axis_index("c")
    base = wid * ROWS_W

    def start_in(i, j):
        r0 = base + i * RB
        pltpu.async_copy(x_hbm.at[:, pl.ds(r0, RB)], xb[j], in_sem[j])
        pltpu.async_copy(pos_hbm.at[pl.ds(r0, RB)], pb[j], pos_sem[j])

    def wait_in(j):
        pltpu.make_async_copy(x_hbm.at[:, pl.ds(base, RB)], xb[j], in_sem[j]).wait()
        pltpu.make_async_copy(pos_hbm.at[pl.ds(base, RB)], pb[j], pos_sem[j]).wait()

    def start_out(i, j):
        r0 = base + i * RB
        pltpu.async_copy(xb[j], out_hbm.at[:, pl.ds(r0, RB)], out_sem[j])

    def wait_out(j):
        pltpu.make_async_copy(xb[j], out_hbm.at[:, pl.ds(base, RB)], out_sem[j]).wait()

    start_in(0, 0)
    start_in(1, 1)

    def outer(ii, carry):
        i0 = ii * RING
        for j in range(RING):
            i = i0 + j
            jp = (j + 2) % RING

            @pl.when(i >= 2)
            def _():
                wait_out(jp)

            @pl.when(i + 2 < NB)
            def _():
                start_in(i + 2, jp)

            wait_in(j)

            def col(c8, cc):
                for u in range(8):
                    sl = pl.ds((c8 * 8 + u) * L, L)
                    for r in range(RB):
                        v = pb[j][r, sl]
                        for b in range(B):
                            plsc.addupdate(xb[j].at[b, r, sl], v)
                return cc

            lax.fori_loop(0, D // L // 8, col, 0)
            start_out(i, j)
        return carry

    lax.fori_loop(0, NB // RING, outer, 0)

    wait_out((NB - 2) % RING)
    wait_out((NB - 1) % RING)


def kernel(x, pos_table):
    return _sc_add(x, pos_table)
